# Initial kernel scaffold; baseline (speedup 1.0000x reference)
import functools

import jax
import jax.numpy as jnp
from jax import lax
from jax.experimental import pallas as pl
from jax.experimental.pallas import tpu as pltpu
from jax.experimental.pallas import tpu_sc as plsc

N_GRAPH = 64
N_NODE = 128
N_FEAT = 9
HIDDEN = 768
LANES = 16
NC = 2
NS = 16
NW = NC * NS
GPW = N_GRAPH // NW
C = 8
NCHUNK = N_NODE // C


def _build_kernel():
    mesh = plsc.VectorSubcoreMesh(core_axis_name="c", subcore_axis_name="s")

    @functools.partial(
        pl.kernel,
        mesh=mesh,
        out_type=jax.ShapeDtypeStruct((N_GRAPH, N_NODE + 1, HIDDEN), jnp.float32),
        scratch_types=[
            pltpu.VMEM((C * N_FEAT,), jnp.int32),
            pltpu.VMEM((C,), jnp.int32),
            pltpu.VMEM((C,), jnp.int32),
            pltpu.VMEM((C * N_FEAT, HIDDEN), jnp.float32),
            pltpu.VMEM((C, HIDDEN), jnp.float32),
            pltpu.VMEM((C, HIDDEN), jnp.float32),
            pltpu.VMEM((C, HIDDEN), jnp.float32),
            pltpu.VMEM((1, HIDDEN), jnp.float32),
        ],
    )
    def k(x_hbm, ind_hbm, outd_hbm, atom_hbm, int_hbm, outt_hbm, tok_hbm,
          out_hbm, xidx, iidx, oidx, arows, irows, orows, res, tokv):
        wid = lax.axis_index("s") * NC + lax.axis_index("c")
        pltpu.sync_copy(tok_hbm, tokv)

        for gl in range(GPW):
            g = wid * GPW + gl
            pltpu.sync_copy(tokv, out_hbm.at[g, pl.ds(0, 1)])

            @pl.loop(0, NCHUNK)
            def _chunk(cix, g=g):
                node0 = cix * C
                pltpu.sync_copy(
                    x_hbm.at[pl.ds(g * (N_NODE * N_FEAT) + node0 * N_FEAT,
                                   C * N_FEAT)],
                    xidx)
                pltpu.sync_copy(ind_hbm.at[pl.ds(g * N_NODE + node0, C)], iidx)
                pltpu.sync_copy(outd_hbm.at[pl.ds(g * N_NODE + node0, C)], oidx)
                pltpu.sync_copy(atom_hbm.at[xidx], arows)
                pltpu.sync_copy(int_hbm.at[iidx], irows)
                pltpu.sync_copy(outt_hbm.at[oidx], orows)

                @pl.loop(0, C)
                def _node(i):
                    @pl.loop(0, HIDDEN // LANES)
                    def _col(j):
                        col = j * LANES
                        acc = arows[i * N_FEAT, pl.ds(col, LANES)]
                        for f in range(1, N_FEAT):
                            acc = acc + arows[i * N_FEAT + f, pl.ds(col, LANES)]
                        acc = acc + irows[i, pl.ds(col, LANES)]
                        acc = acc + orows[i, pl.ds(col, LANES)]
                        res[i, pl.ds(col, LANES)] = acc

                pltpu.sync_copy(res, out_hbm.at[g, pl.ds(1 + node0, C)])

    return k


_KERNEL = _build_kernel()


def kernel(x, in_degree, out_degree, atom_table, in_deg_table, out_deg_table,
           graph_token):
    x = x.reshape(-1).astype(jnp.int32)
    ind = in_degree.reshape(-1).astype(jnp.int32)
    outd = out_degree.reshape(-1).astype(jnp.int32)
    return _KERNEL(x, ind, outd, atom_table, in_deg_table, out_deg_table,
                   graph_token)


# speedup vs baseline: 2.1581x; 2.1581x over previous
import functools

import jax
import jax.numpy as jnp
from jax import lax
from jax.experimental import pallas as pl
from jax.experimental.pallas import tpu as pltpu
from jax.experimental.pallas import tpu_sc as plsc

N_GRAPH = 64
N_NODE = 128
N_FEAT = 9
HIDDEN = 768
LANES = 16
NC = 2
NS = 16
NW = NC * NS
GPW = N_GRAPH // NW
C = 8
NCHUNK = N_NODE // C


def _build_kernel():
    mesh = plsc.VectorSubcoreMesh(core_axis_name="c", subcore_axis_name="s")

    @functools.partial(
        pl.kernel,
        mesh=mesh,
        compiler_params=pltpu.CompilerParams(use_tc_tiling_on_sc=False),
        out_type=jax.ShapeDtypeStruct((N_GRAPH, N_NODE + 1, HIDDEN), jnp.float32),
        scratch_types=[
            pltpu.VMEM((C * N_FEAT,), jnp.int32),
            pltpu.VMEM((C,), jnp.int32),
            pltpu.VMEM((C,), jnp.int32),
            pltpu.VMEM((C * N_FEAT, HIDDEN), jnp.float32),
            pltpu.VMEM((C, HIDDEN), jnp.float32),
            pltpu.VMEM((C, HIDDEN), jnp.float32),
            pltpu.VMEM((C, HIDDEN), jnp.float32),
            pltpu.VMEM((1, HIDDEN), jnp.float32),
        ],
    )
    def k(x_hbm, ind_hbm, outd_hbm, atom_hbm, int_hbm, outt_hbm, tok_hbm,
          out_hbm, xidx, iidx, oidx, arows, irows, orows, res, tokv):
        wid = lax.axis_index("s") * NC + lax.axis_index("c")
        pltpu.sync_copy(tok_hbm, tokv)

        for gl in range(GPW):
            g = wid * GPW + gl
            pltpu.sync_copy(tokv, out_hbm.at[g, pl.ds(0, 1)])

            @pl.loop(0, NCHUNK)
            def _chunk(cix, g=g):
                node0 = cix * C
                pltpu.sync_copy(
                    x_hbm.at[pl.ds(g * (N_NODE * N_FEAT) + node0 * N_FEAT,
                                   C * N_FEAT)],
                    xidx)
                pltpu.sync_copy(ind_hbm.at[pl.ds(g * N_NODE + node0, C)], iidx)
                pltpu.sync_copy(outd_hbm.at[pl.ds(g * N_NODE + node0, C)], oidx)
                pltpu.sync_copy(atom_hbm.at[xidx], arows)
                pltpu.sync_copy(int_hbm.at[iidx], irows)
                pltpu.sync_copy(outt_hbm.at[oidx], orows)

                @pl.loop(0, C)
                def _node(i):
                    @pl.loop(0, HIDDEN // LANES)
                    def _col(j):
                        col = j * LANES
                        acc = arows[i * N_FEAT, pl.ds(col, LANES)]
                        for f in range(1, N_FEAT):
                            acc = acc + arows[i * N_FEAT + f, pl.ds(col, LANES)]
                        acc = acc + irows[i, pl.ds(col, LANES)]
                        acc = acc + orows[i, pl.ds(col, LANES)]
                        res[i, pl.ds(col, LANES)] = acc

                pltpu.sync_copy(res, out_hbm.at[g, pl.ds(1 + node0, C)])

    return k


_KERNEL = _build_kernel()


def kernel(x, in_degree, out_degree, atom_table, in_deg_table, out_deg_table,
           graph_token):
    x = x.reshape(-1).astype(jnp.int32)
    ind = in_degree.reshape(-1).astype(jnp.int32)
    outd = out_degree.reshape(-1).astype(jnp.int32)
    return _KERNEL(x, ind, outd, atom_table, in_deg_table, out_deg_table,
                   graph_token)
